# Initial kernel scaffold; baseline (speedup 1.0000x reference)
import jax
import jax.numpy as jnp
from jax.experimental import pallas as pl


def kernel(pred_raw, u_c, theta_c, F_ext, F_c, L_c, bc_disp, connectivity, elem_lengths, prop_E, prop_A, prop_I22, elem_directions):
    raise NotImplementedError("write your pallas kernel here")


# speedup vs baseline: 28.7718x; 28.7718x over previous
import functools

import jax
import jax.numpy as jnp
from jax import lax
from jax.experimental import pallas as pl
from jax.experimental.pallas import tpu as pltpu
from jax.experimental.pallas import tpu_sc as plsc

N_LANES = 16
NC = 2
NS = 16
SUB = 128
NODE_ALIGN = 512


def _plan(n_elem):
  c = SUB
  for cand in (2560, 1280, 640, 256, 128):
    if NS * cand <= max(n_elem, NS * SUB):
      c = cand
      break
  chunks = max(1, -(-n_elem // (NS * c)))
  return c, chunks, NS * c * chunks


def _sc_pass(pred8, scal16, nAf, nBf, Lp, Ep, Ap, Ip, cp, sp_, fxp, fzp, fmp,
             bcp, npad, c_sz, chunks):
  nsub = c_sz // SUB
  iters = c_sz // N_LANES
  per_sub = chunks * c_sz
  rows_per_sub = npad // 2 // NS
  half = npad // NC
  falloc = half // 2 + SUB
  fz_per_sub = falloc // NS
  trash = half // 2
  nodes_per_sub = half // NS
  node_iters = nodes_per_sub // N_LANES

  mesh = plsc.VectorSubcoreMesh(core_axis_name="c", subcore_axis_name="s",
                                num_cores=NC, num_subcores=NS)

  @functools.partial(
      pl.kernel,
      out_type=jax.ShapeDtypeStruct((NC * NS, 5, N_LANES), jnp.float32),
      mesh=mesh,
      compiler_params=pltpu.CompilerParams(needs_layout_passes=False,
                                           use_tc_tiling_on_sc=False),
      scratch_types=[
          pltpu.VMEM((SUB,), jnp.int32),
          pltpu.VMEM((SUB,), jnp.int32),
          pltpu.VMEM((c_sz,), jnp.float32),
          pltpu.VMEM((c_sz,), jnp.float32),
          pltpu.VMEM((c_sz,), jnp.float32),
          pltpu.VMEM((c_sz,), jnp.float32),
          pltpu.VMEM((c_sz,), jnp.float32),
          pltpu.VMEM((c_sz,), jnp.float32),
          pltpu.VMEM((c_sz, 8), jnp.float32),
          pltpu.VMEM((c_sz, 8), jnp.float32),
          pltpu.VMEM((c_sz, 8), jnp.float32),
          pltpu.VMEM((c_sz, 8), jnp.float32),
          pltpu.VMEM((c_sz,), jnp.int32),
          pltpu.VMEM((c_sz,), jnp.int32),
          pltpu.VMEM((N_LANES,), jnp.float32),
          pltpu.VMEM((N_LANES,), jnp.float32),
          pltpu.VMEM((N_LANES,), jnp.float32),
          pltpu.VMEM((N_LANES,), jnp.float32),
          pltpu.VMEM((N_LANES,), jnp.float32),
          pltpu.VMEM((N_LANES,), jnp.float32),
          pltpu.VMEM((N_LANES,), jnp.float32),
          pltpu.VMEM((N_LANES,), jnp.float32),
          pltpu.VMEM((N_LANES,), jnp.float32),
          pltpu.VMEM_SHARED((NC * falloc, 8), jnp.float32),
          pltpu.SemaphoreType.DMA,
      ],
  )
  def kern(pred8_h, uc_h, tc_h, rfc_h, rfcl_h, nA_h, nB_h, L_h, E_h, A_h,
           I_h, c_h, s_h, fx_h, fz_h, fm_h, bc_h, uout_h,
           nA_v, nB_v, L_v, E_v, A_v, I_v, c_v, s_v,
           uA_r, uB_r, fA_r, fB_r, pA_v, pB_v, uc_v, tc_v, rfc_v, rfcl_v,
           acc0, acc1, acc2, acc3, acc4, f_sp, dsem):
    cid = lax.axis_index("c")
    sid = lax.axis_index("s")
    wid = sid * NC + cid

    pltpu.sync_copy(uc_h, uc_v)
    pltpu.sync_copy(tc_h, tc_v)
    pltpu.sync_copy(rfc_h, rfc_v)
    pltpu.sync_copy(rfcl_h, rfcl_v)

    iota = lax.iota(jnp.int32, N_LANES)
    uc = uc_v[...]
    tc = tc_v[...]
    rfc = rfc_v[...]
    rfcl = rfcl_v[...]
    zero16 = jnp.zeros((N_LANES,), jnp.float32)
    emask = jnp.where(jnp.broadcast_to(cid, (N_LANES,)) == 0, 1.0, 0.0)

    def zbody(i, _):
      idx = i * N_LANES + iota
      zrow = lax.shift_right_logical(idx, 3)
      zcol = lax.bitwise_and(idx, 7)
      plsc.store_scatter(fA_r, [zrow, zcol], zero16)
      plsc.store_scatter(fB_r, [zrow, zcol], zero16)
      return _
    lax.fori_loop(0, c_sz * 8 // N_LANES, zbody, 0)
    pltpu.sync_copy(fA_r.at[pl.ds(0, fz_per_sub)],
                    f_sp.at[pl.ds(sid * fz_per_sub, fz_per_sub)])

    acc0[...] = zero16
    acc1[...] = zero16
    acc2[...] = zero16
    acc3[...] = zero16
    acc4[...] = zero16

    plsc.subcore_barrier()

    h0 = cid * half
    tbase = sid * per_sub

    def chunk_body(k, carry):
      base = tbase + k * c_sz
      pltpu.sync_copy(L_h.at[pl.ds(base, c_sz)], L_v)
      pltpu.sync_copy(E_h.at[pl.ds(base, c_sz)], E_v)
      pltpu.sync_copy(A_h.at[pl.ds(base, c_sz)], A_v)
      pltpu.sync_copy(I_h.at[pl.ds(base, c_sz)], I_v)
      pltpu.sync_copy(c_h.at[pl.ds(base, c_sz)], c_v)
      pltpu.sync_copy(s_h.at[pl.ds(base, c_sz)], s_v)

      def gbody(j, _):
        pltpu.sync_copy(nA_h.at[pl.ds(base + j * SUB, SUB)], nA_v)
        pltpu.sync_copy(nB_h.at[pl.ds(base + j * SUB, SUB)], nB_v)

        def pbody(q, _q):
          qs = pl.ds(q * N_LANES, N_LANES)
          ps = pl.ds(j * SUB + q * N_LANES, N_LANES)
          va = nA_v[qs]
          vb = nB_v[qs]
          pA_v[ps] = lax.bitwise_and(va, 1) * 4
          pB_v[ps] = lax.bitwise_and(vb, 1) * 4
          nA_v[qs] = lax.shift_right_logical(va, 1)
          nB_v[qs] = lax.shift_right_logical(vb, 1)
          return _q
        lax.fori_loop(0, SUB // N_LANES, pbody, 0)
        pltpu.async_copy(pred8_h.at[nA_v], uA_r.at[pl.ds(j * SUB, SUB)],
                         dsem).wait()
        pltpu.async_copy(pred8_h.at[nB_v], uB_r.at[pl.ds(j * SUB, SUB)],
                         dsem).wait()
        return _
      lax.fori_loop(0, nsub, gbody, 0)

      def cbody(i, _):
        row = i * N_LANES + iota
        sl = pl.ds(i * N_LANES, N_LANES)
        Lv = L_v[sl]
        Ev = E_v[sl]
        Av = A_v[sl]
        Iv = I_v[sl]
        cv = c_v[sl]
        sv = s_v[sl]
        parA = pA_v[sl]
        parB = pB_v[sl]
        pxA = plsc.load_gather(uA_r, [row, parA])
        pzA = plsc.load_gather(uA_r, [row, parA + 1])
        ptA = plsc.load_gather(uA_r, [row, parA + 2])
        pxB = plsc.load_gather(uB_r, [row, parB])
        pzB = plsc.load_gather(uB_r, [row, parB + 1])
        ptB = plsc.load_gather(uB_r, [row, parB + 2])
        rL = 1.0 / Lv
        ea_L = Ev * Av * rL
        ei_L = Ev * Iv * rL
        ei_L2 = ei_L * rL
        ei_L3 = ei_L2 * rL
        dux = (pxA - pxB) * uc
        duz = (pzA - pzB) * uc
        du = cv * dux + sv * duz
        dw = cv * duz - sv * dux
        d2 = -(tc * ptA)
        d5 = -(tc * ptB)
        f0 = ea_L * du
        dwb = ei_L2 * dw
        f1 = 12.0 * (ei_L3 * dw) + 6.0 * (ei_L2 * (d2 + d5))
        f2 = 6.0 * dwb + ei_L * (4.0 * d2 + 2.0 * d5)
        f5 = 6.0 * dwb + ei_L * (2.0 * d2 + 4.0 * d5)
        fAx = cv * f0 - sv * f1
        fAz = sv * f0 + cv * f1
        e = f0 * du + f1 * dw + f2 * d2 + f5 * d5
        acc0[...] = acc0[...] + e * emask
        oppA = 4 - parA
        oppB = 4 - parB
        plsc.store_scatter(fA_r, [row, parA], fAx)
        plsc.store_scatter(fA_r, [row, parA + 1], fAz)
        plsc.store_scatter(fA_r, [row, parA + 2], -f2)
        plsc.store_scatter(fA_r, [row, oppA], zero16)
        plsc.store_scatter(fA_r, [row, oppA + 1], zero16)
        plsc.store_scatter(fA_r, [row, oppA + 2], zero16)
        plsc.store_scatter(fB_r, [row, parB], -fAx)
        plsc.store_scatter(fB_r, [row, parB + 1], -fAz)
        plsc.store_scatter(fB_r, [row, parB + 2], -f5)
        plsc.store_scatter(fB_r, [row, oppB], zero16)
        plsc.store_scatter(fB_r, [row, oppB + 1], zero16)
        plsc.store_scatter(fB_r, [row, oppB + 2], zero16)
        return _
      lax.fori_loop(0, iters, cbody, 0)

      def sbody(j, _):
        pltpu.sync_copy(nA_h.at[pl.ds(base + j * SUB, SUB)], nA_v)
        pltpu.sync_copy(nB_h.at[pl.ds(base + j * SUB, SUB)], nB_v)

        def tbody(q, _q):
          qs = pl.ds(q * N_LANES, N_LANES)
          va = nA_v[qs] - h0
          vb = nB_v[qs] - h0
          oka = (va >= 0) & (va < half)
          okb = (vb >= 0) & (vb < half)
          ra = lax.shift_right_logical(va, 1)
          rb = lax.shift_right_logical(vb, 1)
          nA_v[qs] = jnp.where(oka, ra, trash)
          nB_v[qs] = jnp.where(okb, rb, trash)
          return _q
        lax.fori_loop(0, SUB // N_LANES, tbody, 0)
        pltpu.sync_copy(fA_r.at[pl.ds(j * SUB, SUB)], f_sp.at[nA_v],
                        add=True)
        pltpu.sync_copy(fB_r.at[pl.ds(j * SUB, SUB)], f_sp.at[nB_v],
                        add=True)
        return _
      lax.fori_loop(0, nsub, sbody, 0)
      return carry

    lax.fori_loop(0, chunks, chunk_body, 0)

    plsc.subcore_barrier()

    n0l = sid * (nodes_per_sub // 2)
    n0g = h0 + sid * nodes_per_sub
    pltpu.sync_copy(fx_h.at[pl.ds(n0g, nodes_per_sub)],
                    L_v.at[pl.ds(0, nodes_per_sub)])
    pltpu.sync_copy(fz_h.at[pl.ds(n0g, nodes_per_sub)],
                    E_v.at[pl.ds(0, nodes_per_sub)])
    pltpu.sync_copy(fm_h.at[pl.ds(n0g, nodes_per_sub)],
                    A_v.at[pl.ds(0, nodes_per_sub)])
    pltpu.sync_copy(bc_h.at[pl.ds(n0g, nodes_per_sub)],
                    I_v.at[pl.ds(0, nodes_per_sub)])
    pltpu.sync_copy(f_sp.at[pl.ds(n0l, nodes_per_sub // 2)],
                    fA_r.at[pl.ds(0, nodes_per_sub // 2)])
    pltpu.sync_copy(pred8_h.at[pl.ds(n0g // 2, nodes_per_sub // 2)],
                    uB_r.at[pl.ds(0, nodes_per_sub // 2)])

    def nbody(i, _):
      row = i * N_LANES + iota
      sl = pl.ds(i * N_LANES, N_LANES)
      fxv = L_v[sl]
      fzv = E_v[sl]
      fmv = A_v[sl]
      bcv = I_v[sl]
      row8 = i * 8 + lax.shift_right_logical(iota, 1)
      colalt = lax.bitwise_and(iota, 1) * 4
      Fx = plsc.load_gather(fA_r, [row8, colalt])
      Fz = plsc.load_gather(fA_r, [row8, colalt + 1])
      Fm = plsc.load_gather(fA_r, [row8, colalt + 2])
      px = plsc.load_gather(uB_r, [row8, colalt])
      pz = plsc.load_gather(uB_r, [row8, colalt + 1])
      pth = plsc.load_gather(uB_r, [row8, colalt + 2])
      rnx = (Fx + fxv) * rfc
      rnz = (Fz + fzv) * rfc
      rnm = (Fm + fmv) * rfcl
      sq = rnx * rnx + rnz * rnz + rnm * rnm
      freev = jnp.where(bcv < 0.5, 1.0, 0.0)
      acc1[...] = acc1[...] + sq * freev
      acc2[...] = acc2[...] + sq
      acc3[...] = acc3[...] + freev
      acc4[...] = acc4[...] + uc * (fxv * px + fzv * pz) + tc * (fmv * pth)
      return _
    lax.fori_loop(0, node_iters, nbody, 0)

    pltpu.sync_copy(acc0, uout_h.at[wid, 0])
    pltpu.sync_copy(acc1, uout_h.at[wid, 1])
    pltpu.sync_copy(acc2, uout_h.at[wid, 2])
    pltpu.sync_copy(acc3, uout_h.at[wid, 3])
    pltpu.sync_copy(acc4, uout_h.at[wid, 4])

  return kern(pred8, scal16[0], scal16[1], scal16[2], scal16[3], nAf, nBf,
              Lp, Ep, Ap, Ip, cp, sp_, fxp, fzp, fmp, bcp)


def _finalize(predT, u_c, theta_c, upart, n_nodes):
  n = predT.shape[1]

  def body(uc_s, tc_s, pred_r, up_r, loss_r, uphys_r, u_r, w_r):
    uc = uc_s[0]
    tc = tc_s[0]
    pred = pred_r[...]
    col = lax.broadcasted_iota(jnp.int32, (3, n), 0)
    scale = jnp.where(col == 2, tc, uc)
    uphys_r[...] = pred * scale
    up = up_r[...]
    e_sum = jnp.sum(up[:, 0, :])
    sfree = jnp.sum(up[:, 1, :])
    sall = jnp.sum(up[:, 2, :])
    cnt = jnp.sum(up[:, 3, :])
    w_sum = jnp.sum(up[:, 4, :])
    denom = cnt * 3.0
    loss_free = sfree / jnp.maximum(denom, 1.0)
    loss_all = sall / (n_nodes * 3.0)
    loss = jnp.where(denom > 0, loss_free, loss_all)
    loss_r[...] = jnp.broadcast_to(loss, (1, 1))
    u_r[...] = jnp.broadcast_to(0.5 * e_sum, (1, 1))
    w_r[...] = jnp.broadcast_to(w_sum, (1, 1))

  smem = pl.BlockSpec(memory_space=pltpu.MemorySpace.SMEM)
  vmem = pl.BlockSpec(memory_space=pltpu.MemorySpace.VMEM)
  out = pl.pallas_call(
      body,
      in_specs=[smem, smem, vmem, vmem],
      out_specs=[vmem, vmem, vmem, vmem],
      out_shape=[
          jax.ShapeDtypeStruct((1, 1), jnp.float32),
          jax.ShapeDtypeStruct((3, n), jnp.float32),
          jax.ShapeDtypeStruct((1, 1), jnp.float32),
          jax.ShapeDtypeStruct((1, 1), jnp.float32),
      ],
  )(u_c, theta_c, predT, upart)
  return out


def kernel(pred_raw, u_c, theta_c, F_ext, F_c, L_c, bc_disp, connectivity,
           elem_lengths, prop_E, prop_A, prop_I22, elem_directions):
  n = pred_raw.shape[0]
  n_elem = connectivity.shape[0]
  npad = -(-n // NODE_ALIGN) * NODE_ALIGN
  c_sz, chunks, epad = _plan(n_elem)

  pad = epad - n_elem
  zi = jnp.zeros((pad,), jnp.int32)
  zf = jnp.zeros((pad,), jnp.float32)
  nAf = jnp.concatenate([connectivity[:, 0], zi])
  nBf = jnp.concatenate([connectivity[:, 1], zi])
  Lp = jnp.concatenate([elem_lengths, zf + 1.0])
  Ep = jnp.concatenate([prop_E, zf])
  Ap = jnp.concatenate([prop_A, zf])
  Ip = jnp.concatenate([prop_I22, zf])
  cp = jnp.concatenate([elem_directions[:, 0], zf])
  sp_ = jnp.concatenate([elem_directions[:, 2], zf])
  pred8 = jnp.zeros((npad, 4), jnp.float32).at[:n, :3].set(pred_raw)
  pred8 = pred8.reshape(npad // 2, 8)
  nzf = jnp.zeros((npad - n,), jnp.float32)
  fxp = jnp.concatenate([F_ext[:, 0], nzf])
  fzp = jnp.concatenate([F_ext[:, 1], nzf])
  fmp = jnp.concatenate([F_ext[:, 2], nzf])
  bcp = jnp.concatenate([bc_disp[:, 0], nzf + 1.0])

  rfc = 1.0 / F_c
  scal4 = jnp.concatenate([u_c, theta_c, rfc, rfc / L_c]).astype(jnp.float32)
  scal16 = jnp.broadcast_to(scal4[:, None], (4, N_LANES))

  upart = _sc_pass(pred8, scal16, nAf, nBf, Lp, Ep, Ap, Ip, cp, sp_, fxp,
                   fzp, fmp, bcp, npad, c_sz, chunks)
  loss2, uphysT, u2, w2 = _finalize(pred_raw.T, u_c, theta_c, upart, n)
  return (loss2.reshape(()), pred_raw, uphysT.T, u2.reshape(()),
          w2.reshape(()))


# speedup vs baseline: 39.1585x; 1.3610x over previous
import functools

import jax
import jax.numpy as jnp
from jax import lax
from jax.experimental import pallas as pl
from jax.experimental.pallas import tpu as pltpu
from jax.experimental.pallas import tpu_sc as plsc

N_LANES = 16
NC = 2
NS = 16
SUB = 128
NODE_ALIGN = 512


def _plan(n_elem):
  c = SUB
  for cand in (2560, 1280, 640, 256, 128):
    if NS * cand <= max(n_elem, NS * SUB):
      c = cand
      break
  chunks = max(1, -(-n_elem // (NS * c)))
  return c, chunks, NS * c * chunks


def _sc_pass(pred8, scal16, nAf, nBf, Lp, Ep, Ap, Ip, cp, sp_, fxp, fzp, fmp,
             bcp, npad, c_sz, chunks):
  nsub = c_sz // SUB
  iters = c_sz // N_LANES
  per_sub = chunks * c_sz
  rows_per_sub = npad // 2 // NS
  half = npad // NC
  falloc = half // 2 + SUB
  fz_per_sub = falloc // NS
  trash = half // 2
  nodes_per_sub = half // NS
  node_iters = nodes_per_sub // N_LANES

  mesh = plsc.VectorSubcoreMesh(core_axis_name="c", subcore_axis_name="s",
                                num_cores=NC, num_subcores=NS)

  @functools.partial(
      pl.kernel,
      out_type=jax.ShapeDtypeStruct((NC * NS, 5, N_LANES), jnp.float32),
      mesh=mesh,
      compiler_params=pltpu.CompilerParams(needs_layout_passes=False,
                                           use_tc_tiling_on_sc=False),
      scratch_types=[
          pltpu.VMEM((c_sz,), jnp.int32),
          pltpu.VMEM((c_sz,), jnp.int32),
          pltpu.VMEM((c_sz,), jnp.int32),
          pltpu.VMEM((c_sz,), jnp.int32),
          pltpu.VMEM((c_sz,), jnp.float32),
          pltpu.VMEM((c_sz,), jnp.float32),
          pltpu.VMEM((c_sz,), jnp.float32),
          pltpu.VMEM((c_sz,), jnp.float32),
          pltpu.VMEM((c_sz,), jnp.float32),
          pltpu.VMEM((c_sz,), jnp.float32),
          pltpu.VMEM((c_sz, 8), jnp.float32),
          pltpu.VMEM((c_sz, 8), jnp.float32),
          pltpu.VMEM((c_sz, 8), jnp.float32),
          pltpu.VMEM((c_sz, 8), jnp.float32),
          pltpu.VMEM((c_sz,), jnp.int32),
          pltpu.VMEM((c_sz,), jnp.int32),
          pltpu.VMEM((N_LANES,), jnp.float32),
          pltpu.VMEM((N_LANES,), jnp.float32),
          pltpu.VMEM((N_LANES,), jnp.float32),
          pltpu.VMEM((N_LANES,), jnp.float32),
          pltpu.VMEM((N_LANES,), jnp.float32),
          pltpu.VMEM((N_LANES,), jnp.float32),
          pltpu.VMEM((N_LANES,), jnp.float32),
          pltpu.VMEM((N_LANES,), jnp.float32),
          pltpu.VMEM((N_LANES,), jnp.float32),
          pltpu.VMEM_SHARED((NC * falloc, 8), jnp.float32),
          pltpu.SemaphoreType.DMA,
      ],
  )
  def kern(pred8_h, uc_h, tc_h, rfc_h, rfcl_h, gA_h, gB_h, pA_h, pB_h,
           sA_h, sB_h, L_h, E_h, A_h, I_h, c_h, s_h, fx_h, fz_h, fm_h, bc_h,
           uout_h,
           gA_v, gB_v, sA_v, sB_v, L_v, E_v, A_v, I_v, c_v, s_v,
           uA_r, uB_r, fA_r, fB_r, pA_v, pB_v, uc_v, tc_v, rfc_v, rfcl_v,
           acc0, acc1, acc2, acc3, acc4, f_sp, dsem):
    cid = lax.axis_index("c")
    sid = lax.axis_index("s")
    wid = sid * NC + cid

    pltpu.sync_copy(uc_h, uc_v)
    pltpu.sync_copy(tc_h, tc_v)
    pltpu.sync_copy(rfc_h, rfc_v)
    pltpu.sync_copy(rfcl_h, rfcl_v)

    iota = lax.iota(jnp.int32, N_LANES)
    uc = uc_v[...]
    tc = tc_v[...]
    rfc = rfc_v[...]
    rfcl = rfcl_v[...]
    zero16 = jnp.zeros((N_LANES,), jnp.float32)
    emask = jnp.where(jnp.broadcast_to(cid, (N_LANES,)) == 0, 1.0, 0.0)

    def zbody(i, _):
      idx = i * N_LANES + iota
      zrow = lax.shift_right_logical(idx, 3)
      zcol = lax.bitwise_and(idx, 7)
      plsc.store_scatter(fA_r, [zrow, zcol], zero16)
      plsc.store_scatter(fB_r, [zrow, zcol], zero16)
      return _
    lax.fori_loop(0, c_sz * 8 // N_LANES, zbody, 0)
    pltpu.sync_copy(fA_r.at[pl.ds(0, fz_per_sub)],
                    f_sp.at[pl.ds(sid * fz_per_sub, fz_per_sub)])

    acc0[...] = zero16
    acc1[...] = zero16
    acc2[...] = zero16
    acc3[...] = zero16
    acc4[...] = zero16

    plsc.subcore_barrier()

    h0 = cid * half
    tbase = sid * per_sub

    def chunk_body(k, carry):
      base = tbase + k * c_sz
      pltpu.sync_copy(L_h.at[pl.ds(base, c_sz)], L_v)
      pltpu.sync_copy(E_h.at[pl.ds(base, c_sz)], E_v)
      pltpu.sync_copy(A_h.at[pl.ds(base, c_sz)], A_v)
      pltpu.sync_copy(I_h.at[pl.ds(base, c_sz)], I_v)
      pltpu.sync_copy(c_h.at[pl.ds(base, c_sz)], c_v)
      pltpu.sync_copy(s_h.at[pl.ds(base, c_sz)], s_v)

      pltpu.sync_copy(gA_h.at[pl.ds(base, c_sz)], gA_v)
      pltpu.sync_copy(gB_h.at[pl.ds(base, c_sz)], gB_v)
      pltpu.sync_copy(pA_h.at[pl.ds(base, c_sz)], pA_v)
      pltpu.sync_copy(pB_h.at[pl.ds(base, c_sz)], pB_v)
      ck = sid * chunks + k
      pltpu.sync_copy(sA_h.at[cid, ck], sA_v)
      pltpu.sync_copy(sB_h.at[cid, ck], sB_v)
      ga = pltpu.async_copy(pred8_h.at[gA_v], uA_r, dsem)
      gb = pltpu.async_copy(pred8_h.at[gB_v], uB_r, dsem)
      ga.wait()
      gb.wait()

      def cbody(i, _):
        row = i * N_LANES + iota
        sl = pl.ds(i * N_LANES, N_LANES)
        Lv = L_v[sl]
        Ev = E_v[sl]
        Av = A_v[sl]
        Iv = I_v[sl]
        cv = c_v[sl]
        sv = s_v[sl]
        parA = pA_v[sl]
        parB = pB_v[sl]
        pxA = plsc.load_gather(uA_r, [row, parA])
        pzA = plsc.load_gather(uA_r, [row, parA + 1])
        ptA = plsc.load_gather(uA_r, [row, parA + 2])
        pxB = plsc.load_gather(uB_r, [row, parB])
        pzB = plsc.load_gather(uB_r, [row, parB + 1])
        ptB = plsc.load_gather(uB_r, [row, parB + 2])
        rL = 1.0 / Lv
        ea_L = Ev * Av * rL
        ei_L = Ev * Iv * rL
        ei_L2 = ei_L * rL
        ei_L3 = ei_L2 * rL
        dux = (pxA - pxB) * uc
        duz = (pzA - pzB) * uc
        du = cv * dux + sv * duz
        dw = cv * duz - sv * dux
        d2 = -(tc * ptA)
        d5 = -(tc * ptB)
        f0 = ea_L * du
        dwb = ei_L2 * dw
        f1 = 12.0 * (ei_L3 * dw) + 6.0 * (ei_L2 * (d2 + d5))
        f2 = 6.0 * dwb + ei_L * (4.0 * d2 + 2.0 * d5)
        f5 = 6.0 * dwb + ei_L * (2.0 * d2 + 4.0 * d5)
        fAx = cv * f0 - sv * f1
        fAz = sv * f0 + cv * f1
        e = f0 * du + f1 * dw + f2 * d2 + f5 * d5
        acc0[...] = acc0[...] + e * emask
        oppA = 4 - parA
        oppB = 4 - parB
        plsc.store_scatter(fA_r, [row, parA], fAx)
        plsc.store_scatter(fA_r, [row, parA + 1], fAz)
        plsc.store_scatter(fA_r, [row, parA + 2], -f2)
        plsc.store_scatter(fA_r, [row, oppA], zero16)
        plsc.store_scatter(fA_r, [row, oppA + 1], zero16)
        plsc.store_scatter(fA_r, [row, oppA + 2], zero16)
        plsc.store_scatter(fB_r, [row, parB], -fAx)
        plsc.store_scatter(fB_r, [row, parB + 1], -fAz)
        plsc.store_scatter(fB_r, [row, parB + 2], -f5)
        plsc.store_scatter(fB_r, [row, oppB], zero16)
        plsc.store_scatter(fB_r, [row, oppB + 1], zero16)
        plsc.store_scatter(fB_r, [row, oppB + 2], zero16)
        return _
      lax.fori_loop(0, iters, cbody, 0)

      pltpu.sync_copy(fA_r, f_sp.at[sA_v], add=True)
      pltpu.sync_copy(fB_r, f_sp.at[sB_v], add=True)
      return carry

    lax.fori_loop(0, chunks, chunk_body, 0)

    plsc.subcore_barrier()

    n0l = sid * (nodes_per_sub // 2)
    n0g = h0 + sid * nodes_per_sub
    pltpu.sync_copy(fx_h.at[pl.ds(n0g, nodes_per_sub)],
                    L_v.at[pl.ds(0, nodes_per_sub)])
    pltpu.sync_copy(fz_h.at[pl.ds(n0g, nodes_per_sub)],
                    E_v.at[pl.ds(0, nodes_per_sub)])
    pltpu.sync_copy(fm_h.at[pl.ds(n0g, nodes_per_sub)],
                    A_v.at[pl.ds(0, nodes_per_sub)])
    pltpu.sync_copy(bc_h.at[pl.ds(n0g, nodes_per_sub)],
                    I_v.at[pl.ds(0, nodes_per_sub)])
    pltpu.sync_copy(f_sp.at[pl.ds(n0l, nodes_per_sub // 2)],
                    fA_r.at[pl.ds(0, nodes_per_sub // 2)])
    pltpu.sync_copy(pred8_h.at[pl.ds(n0g // 2, nodes_per_sub // 2)],
                    uB_r.at[pl.ds(0, nodes_per_sub // 2)])

    def nbody(i, _):
      row = i * N_LANES + iota
      sl = pl.ds(i * N_LANES, N_LANES)
      fxv = L_v[sl]
      fzv = E_v[sl]
      fmv = A_v[sl]
      bcv = I_v[sl]
      row8 = i * 8 + lax.shift_right_logical(iota, 1)
      colalt = lax.bitwise_and(iota, 1) * 4
      Fx = plsc.load_gather(fA_r, [row8, colalt])
      Fz = plsc.load_gather(fA_r, [row8, colalt + 1])
      Fm = plsc.load_gather(fA_r, [row8, colalt + 2])
      px = plsc.load_gather(uB_r, [row8, colalt])
      pz = plsc.load_gather(uB_r, [row8, colalt + 1])
      pth = plsc.load_gather(uB_r, [row8, colalt + 2])
      rnx = (Fx + fxv) * rfc
      rnz = (Fz + fzv) * rfc
      rnm = (Fm + fmv) * rfcl
      sq = rnx * rnx + rnz * rnz + rnm * rnm
      freev = jnp.where(bcv < 0.5, 1.0, 0.0)
      acc1[...] = acc1[...] + sq * freev
      acc2[...] = acc2[...] + sq
      acc3[...] = acc3[...] + freev
      acc4[...] = acc4[...] + uc * (fxv * px + fzv * pz) + tc * (fmv * pth)
      return _
    lax.fori_loop(0, node_iters, nbody, 0)

    pltpu.sync_copy(acc0, uout_h.at[wid, 0])
    pltpu.sync_copy(acc1, uout_h.at[wid, 1])
    pltpu.sync_copy(acc2, uout_h.at[wid, 2])
    pltpu.sync_copy(acc3, uout_h.at[wid, 3])
    pltpu.sync_copy(acc4, uout_h.at[wid, 4])

  gA = lax.shift_right_logical(nAf, 1)
  gB = lax.shift_right_logical(nBf, 1)
  pA4 = lax.bitwise_and(nAf, 1) * 4
  pB4 = lax.bitwise_and(nBf, 1) * 4
  scs = []
  for node_ids in (nAf, nBf):
    percore = []
    for c in range(NC):
      loc = node_ids - c * half
      ok = (loc >= 0) & (loc < half)
      percore.append(jnp.where(ok, lax.shift_right_logical(loc, 1), trash))
    scs.append(jnp.stack(percore).reshape(NC, NS * chunks, c_sz))
  return kern(pred8, scal16[0], scal16[1], scal16[2], scal16[3], gA, gB,
              pA4, pB4, scs[0], scs[1], Lp, Ep, Ap, Ip, cp, sp_, fxp, fzp,
              fmp, bcp)


def _finalize(predT, u_c, theta_c, upart, n_nodes):
  n = predT.shape[1]

  def body(uc_s, tc_s, pred_r, up_r, loss_r, uphys_r, u_r, w_r):
    uc = uc_s[0]
    tc = tc_s[0]
    pred = pred_r[...]
    col = lax.broadcasted_iota(jnp.int32, (3, n), 0)
    scale = jnp.where(col == 2, tc, uc)
    uphys_r[...] = pred * scale
    up = up_r[...]
    e_sum = jnp.sum(up[:, 0, :])
    sfree = jnp.sum(up[:, 1, :])
    sall = jnp.sum(up[:, 2, :])
    cnt = jnp.sum(up[:, 3, :])
    w_sum = jnp.sum(up[:, 4, :])
    denom = cnt * 3.0
    loss_free = sfree / jnp.maximum(denom, 1.0)
    loss_all = sall / (n_nodes * 3.0)
    loss = jnp.where(denom > 0, loss_free, loss_all)
    loss_r[...] = jnp.broadcast_to(loss, (1, 1))
    u_r[...] = jnp.broadcast_to(0.5 * e_sum, (1, 1))
    w_r[...] = jnp.broadcast_to(w_sum, (1, 1))

  smem = pl.BlockSpec(memory_space=pltpu.MemorySpace.SMEM)
  vmem = pl.BlockSpec(memory_space=pltpu.MemorySpace.VMEM)
  out = pl.pallas_call(
      body,
      in_specs=[smem, smem, vmem, vmem],
      out_specs=[vmem, vmem, vmem, vmem],
      out_shape=[
          jax.ShapeDtypeStruct((1, 1), jnp.float32),
          jax.ShapeDtypeStruct((3, n), jnp.float32),
          jax.ShapeDtypeStruct((1, 1), jnp.float32),
          jax.ShapeDtypeStruct((1, 1), jnp.float32),
      ],
  )(u_c, theta_c, predT, upart)
  return out


def kernel(pred_raw, u_c, theta_c, F_ext, F_c, L_c, bc_disp, connectivity,
           elem_lengths, prop_E, prop_A, prop_I22, elem_directions):
  n = pred_raw.shape[0]
  n_elem = connectivity.shape[0]
  npad = -(-n // NODE_ALIGN) * NODE_ALIGN
  c_sz, chunks, epad = _plan(n_elem)

  pad = epad - n_elem
  zi = jnp.zeros((pad,), jnp.int32)
  zf = jnp.zeros((pad,), jnp.float32)
  nAf = jnp.concatenate([connectivity[:, 0], zi])
  nBf = jnp.concatenate([connectivity[:, 1], zi])
  Lp = jnp.concatenate([elem_lengths, zf + 1.0])
  Ep = jnp.concatenate([prop_E, zf])
  Ap = jnp.concatenate([prop_A, zf])
  Ip = jnp.concatenate([prop_I22, zf])
  cp = jnp.concatenate([elem_directions[:, 0], zf])
  sp_ = jnp.concatenate([elem_directions[:, 2], zf])
  pred8 = jnp.zeros((npad, 4), jnp.float32).at[:n, :3].set(pred_raw)
  pred8 = pred8.reshape(npad // 2, 8)
  nzf = jnp.zeros((npad - n,), jnp.float32)
  fxp = jnp.concatenate([F_ext[:, 0], nzf])
  fzp = jnp.concatenate([F_ext[:, 1], nzf])
  fmp = jnp.concatenate([F_ext[:, 2], nzf])
  bcp = jnp.concatenate([bc_disp[:, 0], nzf + 1.0])

  rfc = 1.0 / F_c
  scal4 = jnp.concatenate([u_c, theta_c, rfc, rfc / L_c]).astype(jnp.float32)
  scal16 = jnp.broadcast_to(scal4[:, None], (4, N_LANES))

  upart = _sc_pass(pred8, scal16, nAf, nBf, Lp, Ep, Ap, Ip, cp, sp_, fxp,
                   fzp, fmp, bcp, npad, c_sz, chunks)
  loss2, uphysT, u2, w2 = _finalize(pred_raw.T, u_c, theta_c, upart, n)
  return (loss2.reshape(()), pred_raw, uphysT.T, u2.reshape(()),
          w2.reshape(()))
